# baseline (speedup 1.0000x reference)
import jax
import jax.numpy as jnp
from jax import lax
from jax.experimental import pallas as pl
from jax.experimental.pallas import tpu as pltpu
from jax.experimental.pallas import tpu_sc as plsc

_B = 16384
_D = 32
_NE = 1000000
_NC = 2
_NS = 16
_DPC = _D // _NC
_IPT = _B // _NS
_NRV = 1000
_RV_LINES = _NRV * _D // 128
_NW = _NC * _NS
_BPW = _B // _NW
_SPLIT_A = 500096
_SPLIT_T = 999936
_LEN_B = _SPLIT_T - _SPLIT_A
_LEN_T = _NE - _SPLIT_T


def _partial(Et_hbm, tailT_hbm, rv_hbm, u_hbm, r_hbm, v_hbm, part_hbm,
             row_a, row_b, row_t, u_idx_v, r_idx_v, v_idx_v,
             u_hA_v, u_hB_v, u_hT_v, v_hA_v, v_hB_v, v_hT_v,
             u_val, v_val, acc, rv_l,
             sem_rv, sem_u, sem_v, sem_sa, sem_sb, sem_st):
    cid = lax.axis_index("c")
    sid = lax.axis_index("s")
    base = sid * _IPT

    crv = pltpu.async_copy(rv_hbm, rv_l, sem_rv)
    pltpu.sync_copy(u_hbm.at[pl.ds(base, _IPT)], u_idx_v)
    pltpu.sync_copy(v_hbm.at[pl.ds(base, _IPT)], v_idx_v)
    pltpu.sync_copy(r_hbm.at[pl.ds(base, _IPT)], r_idx_v)

    for s in range(_IPT // 16):
        sl = pl.ds(s * 16, 16)
        neg1 = jnp.full((16,), -1, jnp.int32)
        loA = jnp.full((16,), _SPLIT_A, jnp.int32)
        loT = jnp.full((16,), _SPLIT_T, jnp.int32)
        for idx_v, hA, hB, hT in ((u_idx_v, u_hA_v, u_hB_v, u_hT_v),
                                  (v_idx_v, v_hA_v, v_hB_v, v_hT_v)):
            i = idx_v[sl]
            hA[sl] = jnp.where(i < loA, i, neg1)
            hB[sl] = jnp.where(jnp.logical_and(i >= loA, i < loT),
                               i - loA, neg1)
            hT[sl] = jnp.where(i >= loT, i - loT, neg1)

    for s in range(_IPT // 16):
        acc[pl.ds(s * 16, 16)] = jnp.zeros((16,), jnp.float32)
    crv.wait()

    three = jnp.full((16,), 3, jnp.int32)

    def fire_stage(d):
        @pl.when(sid == 0)
        def _stage():
            pltpu.async_copy(Et_hbm.at[d].at[pl.ds(0, _SPLIT_A)],
                             row_a, sem_sa)
            pltpu.async_copy(Et_hbm.at[d].at[pl.ds(_SPLIT_A, _LEN_B)],
                             row_b, sem_sb)
            pltpu.async_copy(tailT_hbm.at[d], row_t, sem_st)

    def wait_stage(d):
        @pl.when(sid == 0)
        def _wait():
            pltpu.make_async_copy(Et_hbm.at[d].at[pl.ds(0, _SPLIT_A)],
                                  row_a, sem_sa).wait()
            pltpu.make_async_copy(Et_hbm.at[d].at[pl.ds(_SPLIT_A, _LEN_B)],
                                  row_b, sem_sb).wait()
            pltpu.make_async_copy(tailT_hbm.at[d], row_t, sem_st).wait()

    fire_stage(cid * _DPC)

    def dim_body(k, carry):
        d = cid * _DPC + k
        wait_stage(d)
        plsc.subcore_barrier()

        cua = pltpu.async_copy(
            row_a.at[plsc.Indices(u_hA_v, ignored_value=-1)], u_val, sem_u)
        cub = pltpu.async_copy(
            row_b.at[plsc.Indices(u_hB_v, ignored_value=-1)], u_val, sem_u)
        cut = pltpu.async_copy(
            row_t.at[plsc.Indices(u_hT_v, ignored_value=-1)], u_val, sem_u)
        cva = pltpu.async_copy(
            row_a.at[plsc.Indices(v_hA_v, ignored_value=-1)], v_val, sem_v)
        cvb = pltpu.async_copy(
            row_b.at[plsc.Indices(v_hB_v, ignored_value=-1)], v_val, sem_v)
        cvt = pltpu.async_copy(
            row_t.at[plsc.Indices(v_hT_v, ignored_value=-1)], v_val, sem_v)
        for c in (cua, cub, cut, cva, cvb, cvt):
            c.wait()
        plsc.subcore_barrier()

        @pl.when(k + 1 < _DPC)
        def _prefetch():
            fire_stage(d + 1)

        def slice_body(s, carry2, d=d):
            sl = pl.ds(s * 16, 16)
            ridx = r_idx_v[sl]
            rline = lax.shift_right_logical(ridx, 2)
            rcol = lax.shift_left(jnp.bitwise_and(ridx, three), 5) + d
            rd = plsc.load_gather(rv_l, [rline, rcol])
            t = u_val[sl] - (v_val[sl] + rd)
            acc[sl] = acc[sl] + t * t
            return carry2

        lax.fori_loop(0, _IPT // 16, slice_body, 0)
        return carry

    lax.fori_loop(0, _DPC, dim_body, 0)

    pltpu.sync_copy(acc, part_hbm.at[cid, pl.ds(base, _IPT)])


def _combine(part_hbm, out_hbm, p0_v, p1_v, out_v):
    wid = lax.axis_index("s") * _NC + lax.axis_index("c")
    base = wid * _BPW
    pltpu.sync_copy(part_hbm.at[0, pl.ds(base, _BPW)], p0_v)
    pltpu.sync_copy(part_hbm.at[1, pl.ds(base, _BPW)], p1_v)
    for s in range(_BPW // 16):
        sl = pl.ds(s * 16, 16)
        out_v[sl] = -(p0_v[sl] + p1_v[sl])
    pltpu.sync_copy(out_v, out_hbm.at[pl.ds(base, _BPW)])


@jax.jit
def kernel(E, rv, u_idx, r_idx, v_idx):
    Et = E.T
    tailT = lax.slice(Et, (0, _SPLIT_T), (_D, _NE))
    rv_lines = rv.reshape(-1, 128)
    mesh = plsc.VectorSubcoreMesh(core_axis_name="c", subcore_axis_name="s")

    partial = pl.kernel(
        _partial,
        out_type=jax.ShapeDtypeStruct((_NC, _B), jnp.float32),
        mesh=mesh,
        compiler_params=pltpu.CompilerParams(needs_layout_passes=False),
        scratch_types=[
            pltpu.VMEM_SHARED((_SPLIT_A,), jnp.float32),
            pltpu.VMEM_SHARED((_LEN_B,), jnp.float32),
            pltpu.VMEM_SHARED((_LEN_T,), jnp.float32),
            pltpu.VMEM((_IPT,), jnp.int32),
            pltpu.VMEM((_IPT,), jnp.int32),
            pltpu.VMEM((_IPT,), jnp.int32),
            pltpu.VMEM((_IPT,), jnp.int32),
            pltpu.VMEM((_IPT,), jnp.int32),
            pltpu.VMEM((_IPT,), jnp.int32),
            pltpu.VMEM((_IPT,), jnp.int32),
            pltpu.VMEM((_IPT,), jnp.int32),
            pltpu.VMEM((_IPT,), jnp.int32),
            pltpu.VMEM((_IPT,), jnp.float32),
            pltpu.VMEM((_IPT,), jnp.float32),
            pltpu.VMEM((_IPT,), jnp.float32),
            pltpu.VMEM((_RV_LINES, 128), jnp.float32),
            pltpu.SemaphoreType.DMA,
            pltpu.SemaphoreType.DMA,
            pltpu.SemaphoreType.DMA,
            pltpu.SemaphoreType.DMA,
            pltpu.SemaphoreType.DMA,
            pltpu.SemaphoreType.DMA,
        ],
    )
    part = partial(Et, tailT, rv_lines, u_idx, r_idx, v_idx)

    combine = pl.kernel(
        _combine,
        out_type=jax.ShapeDtypeStruct((_B,), jnp.float32),
        mesh=mesh,
        compiler_params=pltpu.CompilerParams(needs_layout_passes=False),
        scratch_types=[
            pltpu.VMEM((_BPW,), jnp.float32),
            pltpu.VMEM((_BPW,), jnp.float32),
            pltpu.VMEM((_BPW,), jnp.float32),
        ],
    )
    return combine(part)


# speedup vs baseline: 1.1465x; 1.1465x over previous
import jax
import jax.numpy as jnp
from jax import lax
from jax.experimental import pallas as pl
from jax.experimental.pallas import tpu as pltpu
from jax.experimental.pallas import tpu_sc as plsc

_B = 16384
_D = 32
_NE = 1000000
_NC = 2
_NS = 16
_DPC = _D // _NC
_IPT = _B // _NS
_NRV = 1000
_RV_LINES = _NRV * _D // 128
_NW = _NC * _NS
_BPW = _B // _NW


def _partial(Et_hbm, rv_hbm, u_hbm, r_hbm, v_hbm, part_hbm,
             row_sh, uv_idx_v, r_idx_v, uv_val, acc, rv_l,
             sem_rv, sem_uv, sem_st):
    cid = lax.axis_index("c")
    sid = lax.axis_index("s")
    base = sid * _IPT

    crv = pltpu.async_copy(rv_hbm, rv_l, sem_rv)
    pltpu.sync_copy(u_hbm.at[pl.ds(base, _IPT)],
                    uv_idx_v.at[pl.ds(0, _IPT)])
    pltpu.sync_copy(v_hbm.at[pl.ds(base, _IPT)],
                    uv_idx_v.at[pl.ds(_IPT, _IPT)])
    pltpu.sync_copy(r_hbm.at[pl.ds(base, _IPT)], r_idx_v)

    for s in range(_IPT // 16):
        acc[pl.ds(s * 16, 16)] = jnp.zeros((16,), jnp.float32)
    crv.wait()

    three = jnp.full((16,), 3, jnp.int32)

    def fire_stage(d):
        @pl.when(sid == 0)
        def _stage():
            pltpu.async_copy(Et_hbm.at[d], row_sh, sem_st)

    def wait_stage(d):
        @pl.when(sid == 0)
        def _wait():
            pltpu.make_async_copy(Et_hbm.at[d], row_sh, sem_st).wait()

    fire_stage(cid * _DPC)

    def dim_body(k, carry):
        d = cid * _DPC + k
        wait_stage(d)
        plsc.subcore_barrier()

        cuv = pltpu.async_copy(row_sh.at[uv_idx_v], uv_val, sem_uv)
        cuv.wait()
        plsc.subcore_barrier()

        @pl.when(k + 1 < _DPC)
        def _prefetch():
            fire_stage(d + 1)

        def slice_body(s, carry2, d=d):
            sl = pl.ds(s * 16, 16)
            ridx = r_idx_v[sl]
            rline = lax.shift_right_logical(ridx, 2)
            rcol = lax.shift_left(jnp.bitwise_and(ridx, three), 5) + d
            rd = plsc.load_gather(rv_l, [rline, rcol])
            t = uv_val[sl] - (uv_val[pl.ds(_IPT + s * 16, 16)] + rd)
            acc[sl] = acc[sl] + t * t
            return carry2

        lax.fori_loop(0, _IPT // 16, slice_body, 0)
        return carry

    lax.fori_loop(0, _DPC, dim_body, 0)

    pltpu.sync_copy(acc, part_hbm.at[cid, pl.ds(base, _IPT)])


def _combine(part_hbm, out_hbm, p0_v, p1_v, out_v):
    wid = lax.axis_index("s") * _NC + lax.axis_index("c")
    base = wid * _BPW
    pltpu.sync_copy(part_hbm.at[0, pl.ds(base, _BPW)], p0_v)
    pltpu.sync_copy(part_hbm.at[1, pl.ds(base, _BPW)], p1_v)
    for s in range(_BPW // 16):
        sl = pl.ds(s * 16, 16)
        out_v[sl] = -(p0_v[sl] + p1_v[sl])
    pltpu.sync_copy(out_v, out_hbm.at[pl.ds(base, _BPW)])


@jax.jit
def kernel(E, rv, u_idx, r_idx, v_idx):
    Et = E.T
    rv_lines = rv.reshape(-1, 128)
    mesh = plsc.VectorSubcoreMesh(core_axis_name="c", subcore_axis_name="s")

    partial = pl.kernel(
        _partial,
        out_type=jax.ShapeDtypeStruct((_NC, _B), jnp.float32),
        mesh=mesh,
        compiler_params=pltpu.CompilerParams(needs_layout_passes=False),
        scratch_types=[
            pltpu.VMEM_SHARED((_NE,), jnp.float32),
            pltpu.VMEM((2 * _IPT,), jnp.int32),
            pltpu.VMEM((_IPT,), jnp.int32),
            pltpu.VMEM((2 * _IPT,), jnp.float32),
            pltpu.VMEM((_IPT,), jnp.float32),
            pltpu.VMEM((_RV_LINES, 128), jnp.float32),
            pltpu.SemaphoreType.DMA,
            pltpu.SemaphoreType.DMA,
            pltpu.SemaphoreType.DMA,
        ],
    )
    part = partial(Et, rv_lines, u_idx, r_idx, v_idx)

    combine = pl.kernel(
        _combine,
        out_type=jax.ShapeDtypeStruct((_B,), jnp.float32),
        mesh=mesh,
        compiler_params=pltpu.CompilerParams(needs_layout_passes=False),
        scratch_types=[
            pltpu.VMEM((_BPW,), jnp.float32),
            pltpu.VMEM((_BPW,), jnp.float32),
            pltpu.VMEM((_BPW,), jnp.float32),
        ],
    )
    return combine(part)
